# baseline (speedup 1.0000x reference)
import jax
import jax.numpy as jnp
from jax import lax
from jax.experimental import pallas as pl
from jax.experimental.pallas import tpu as pltpu
from jax.experimental.pallas import tpu_sc as plsc

CAP = 1_000_000
DIM = 32
N_ADD = 16384
N_SAMPLE = 4096
L = 16
NW = 32
RANGE = CAP // NW
TBL = RANGE + (-RANGE) % L
DUMMY = N_SAMPLE
WIN = 128
GRP = 8
SCH = 512
LIST = N_SAMPLE + L
TAILBASE = (CAP // WIN) * WIN
TAILSTART = CAP - WIN


def _bcast(x, r):
    return x.at[jnp.full((L,), r, jnp.int32)].get(mode="promise_in_bounds")


def _body(mem_t, idx_h, val_t, smp_h, tail_t, out_h,
          idx_v, smp_v, tbl_v, bad_v, mj_v, mo_v, us_v, uo_v,
          tb_v, rows_v, sem, sem2):
    cid = lax.axis_index("c")
    sid = lax.axis_index("s")
    wid = sid * 2 + cid
    lo = wid * RANGE
    iota = lax.iota(jnp.int32, L)
    zero_v = jnp.zeros((L,), jnp.int32)

    pltpu.sync_copy(idx_h, idx_v)
    pltpu.sync_copy(smp_h, smp_v)

    neg1 = jnp.full((L,), -1, jnp.int32)

    def init_body(i, c):
        for u in range(8):
            tbl_v[pl.ds(i * (8 * L) + u * L, L)] = neg1
        return c

    lax.fori_loop(0, TBL // (8 * L), init_body, 0)
    for u in range((TBL % (8 * L)) // L):
        tbl_v[pl.ds(TBL - (TBL % (8 * L)) + u * L, L)] = neg1

    dummy = jnp.full((L,), DUMMY, jnp.int32)

    def pre_body(i, c):
        s = pl.ds(i * L, L)
        mj_v[s] = zero_v
        mo_v[s] = dummy
        us_v[s] = zero_v
        uo_v[s] = dummy
        return c

    lax.fori_loop(0, LIST // L, pre_body, 0)

    def scat_body(i, nbad):
        v = idx_v[pl.ds(i * L, L)]
        m = (v >= lo) & (v < lo + RANGE)
        t = jnp.where(m, v - lo, 0)
        j = i * L + iota
        plsc.store_scatter(tbl_v, [t], j, mask=m)
        g = plsc.load_gather(tbl_v, [t])
        flag = m & (g < j)
        fi = jnp.where(flag, 1, 0).astype(jnp.int32)
        pos = nbad + plsc.cumsum(fi) - 1
        posc = jnp.where(flag, pos, 0)
        packed = t * N_ADD + j
        plsc.store_scatter(bad_v, [posc], packed, mask=flag)
        return nbad + plsc.all_reduce_population_count(flag)

    nbad_v = lax.fori_loop(0, N_ADD // L, scat_body, zero_v, unroll=4)
    nbad = jnp.max(nbad_v)

    def fix_body(i, c):
        chunk = lax.shift_right_logical(i, 4)
        lane = lax.bitwise_and(i, L - 1)
        pk = bad_v[pl.ds(chunk * L, L)]
        vp = lax.shift_right_logical(pk, 14)
        vj = lax.bitwise_and(pk, N_ADD - 1)
        onemask = iota == lane
        g = plsc.load_gather(tbl_v, [vp])
        plsc.store_scatter(tbl_v, [vp], jnp.maximum(g, vj), mask=onemask)
        return c

    lax.fori_loop(0, nbad, fix_body, 0)

    def smp_body(i, offs):
        om, ou, ot = offs
        sv = smp_v[pl.ds(i * L, L)]
        m = (sv >= lo) & (sv < lo + RANGE)
        t = jnp.where(m, sv - lo, 0)
        g = plsc.load_gather(tbl_v, [t])
        mm = m & (g >= 0)
        mru = m & (g < 0)
        mt = mru & (sv >= TAILBASE)
        mu = mru & (sv < TAILBASE)
        spos = i * L + iota
        im = jnp.where(mm, 1, 0).astype(jnp.int32)
        iu = jnp.where(mu, 1, 0).astype(jnp.int32)
        it = jnp.where(mt, 1, 0).astype(jnp.int32)
        pm = om + plsc.cumsum(im) - 1
        pu = ou + plsc.cumsum(iu) - 1
        pt = (N_SAMPLE - 1) - (ot + plsc.cumsum(it) - 1)
        pmc = jnp.where(mm, pm, 0)
        puc = jnp.where(mu, pu, 0)
        ptc = jnp.where(mt, pt, 0)
        plsc.store_scatter(mj_v, [pmc], g, mask=mm)
        plsc.store_scatter(mo_v, [pmc], spos, mask=mm)
        plsc.store_scatter(us_v, [puc], sv, mask=mu)
        plsc.store_scatter(uo_v, [puc], spos, mask=mu)
        plsc.store_scatter(us_v, [ptc], sv, mask=mt)
        plsc.store_scatter(uo_v, [ptc], spos, mask=mt)
        return (om + plsc.all_reduce_population_count(mm),
                ou + plsc.all_reduce_population_count(mu),
                ot + plsc.all_reduce_population_count(mt))

    nm_v, nu_v, nt_v = lax.fori_loop(0, N_SAMPLE // L, smp_body,
                                     (zero_v, zero_v, zero_v), unroll=2)
    nm = jnp.max(nm_v)
    nu = jnp.max(nu_v)
    nt = jnp.max(nt_v)

    def assemble(b, lane_splat):
        bsel = jnp.full((L,), b, jnp.int32)
        g0 = plsc.load_gather(tb_v, [bsel, iota, lane_splat])
        g1 = plsc.load_gather(tb_v, [bsel, iota + L, lane_splat])
        rows_v[b, pl.ds(0, L)] = g0
        rows_v[b, pl.ds(L, L)] = g1

    def move_rows(src_list, dst_list, src_t, n):
        def group_body(g, carry):
            sv16 = src_list[pl.ds(g * GRP, L)]
            svals, descs = [], []
            for b in range(GRP):
                s = jnp.max(jnp.where(iota == b, sv16, 0))
                svals.append(s)
                ofs = pl.multiple_of(
                    lax.shift_left(lax.shift_right_logical(s, 7), 7), WIN)
                for ci in range(4):
                    descs.append(pltpu.async_copy(
                        src_t.at[pl.ds(ci * 8, 8), pl.ds(ofs, WIN)],
                        tb_v.at[b, pl.ds(ci * 8, 8)], sem))
            dv = dst_list[pl.ds(g * GRP, L)]
            dvx = jnp.where(iota < GRP, dv, DUMMY)
            for d in descs:
                d.wait()
            for b in range(GRP):
                lane = jnp.zeros((L,), jnp.int32) + lax.bitwise_and(
                    svals[b], WIN - 1)
                assemble(b, lane)
            pltpu.async_copy(rows_v, out_h.at[dvx], sem2).wait()
            return carry

        lax.fori_loop(0, (n + GRP - 1) // GRP, group_body, 0)

    move_rows(mj_v, mo_v, val_t, nm)
    move_rows(us_v, uo_v, mem_t, nu)

    pltpu.async_copy(tail_t, tb_v.at[0], sem).wait()

    def tail_body(i, carry):
        e = (N_SAMPLE - 1) - i
        ch = lax.shift_right_logical(e, 4)
        ln = lax.bitwise_and(e, L - 1)
        sv = us_v[pl.ds(ch * L, L)]
        dvv = uo_v[pl.ds(ch * L, L)]
        s_spl = _bcast(sv, ln)
        d_spl = _bcast(dvv, ln)
        lane = s_spl - TAILSTART
        bsel = jnp.zeros((L,), jnp.int32)
        g0 = plsc.load_gather(tb_v, [bsel, iota, lane])
        g1 = plsc.load_gather(tb_v, [bsel, iota + L, lane])
        rows_v[0, pl.ds(0, L)] = g0
        rows_v[0, pl.ds(L, L)] = g1
        dvx = jnp.where(iota == 0, d_spl, DUMMY)
        pltpu.async_copy(rows_v, out_h.at[dvx], sem2).wait()
        return carry

    lax.fori_loop(0, nt, tail_body, 0)


_sc_call_cache = []


def _get_sc_call():
    if not _sc_call_cache:
        _sc_call_cache.append(_build_sc_call())
    return _sc_call_cache[0]


def _build_sc_call():
    return pl.kernel(
        _body,
        out_type=jax.ShapeDtypeStruct((N_SAMPLE + 8, 128), jnp.float32),
        mesh=plsc.VectorSubcoreMesh(core_axis_name="c", subcore_axis_name="s"),
        compiler_params=pltpu.CompilerParams(needs_layout_passes=False),
        scratch_types=[
            pltpu.VMEM((N_ADD,), jnp.int32),
            pltpu.VMEM((N_SAMPLE,), jnp.int32),
            pltpu.VMEM((TBL,), jnp.int32),
            pltpu.VMEM((N_ADD,), jnp.int32),
            pltpu.VMEM((LIST,), jnp.int32),
            pltpu.VMEM((LIST,), jnp.int32),
            pltpu.VMEM((LIST,), jnp.int32),
            pltpu.VMEM((LIST,), jnp.int32),
            pltpu.VMEM((GRP, DIM, WIN), jnp.float32),
            pltpu.VMEM((L, 128), jnp.float32),
            pltpu.SemaphoreType.DMA,
            pltpu.SemaphoreType.DMA,
        ],
    )


def kernel(mem, idx, val, sample_idx):
    mem_t = mem.T
    tail_t = mem_t[:, TAILSTART:]
    out = _get_sc_call()(mem_t, idx, val.T, sample_idx, tail_t)
    return out[:N_SAMPLE, :DIM]


# speedup vs baseline: 1.0009x; 1.0009x over previous
import jax
import jax.numpy as jnp
from jax import lax
from jax.experimental import pallas as pl
from jax.experimental.pallas import tpu as pltpu
from jax.experimental.pallas import tpu_sc as plsc

CAP = 1_000_000
DIM = 32
N_ADD = 16384
N_SAMPLE = 4096
L = 16
NW = 32
RANGE = CAP // NW
TBL = RANGE + (-RANGE) % L
DUMMY = N_SAMPLE
WIN = 128
GRP = 8
SCH = 512
LIST = N_SAMPLE + L
TAILBASE = (CAP // WIN) * WIN
TAILSTART = CAP - WIN


def _bcast(x, r):
    return x.at[jnp.full((L,), r, jnp.int32)].get(mode="promise_in_bounds")


def _body(mem_t, idx_h, val_t, smp_h, tail_t, out_h,
          idx_v, smp_v, tbl_v, bad_v, mj_v, mo_v, us_v, uo_v,
          tb_v, rows_v, sem, sem2):
    cid = lax.axis_index("c")
    sid = lax.axis_index("s")
    wid = sid * 2 + cid
    lo = wid * RANGE
    iota = lax.iota(jnp.int32, L)
    zero_v = jnp.zeros((L,), jnp.int32)

    pltpu.sync_copy(idx_h, idx_v)
    pltpu.sync_copy(smp_h, smp_v)

    neg1 = jnp.full((L,), -1, jnp.int32)

    def init_body(i, c):
        for u in range(8):
            tbl_v[pl.ds(i * (8 * L) + u * L, L)] = neg1
        return c

    lax.fori_loop(0, TBL // (8 * L), init_body, 0)
    for u in range((TBL % (8 * L)) // L):
        tbl_v[pl.ds(TBL - (TBL % (8 * L)) + u * L, L)] = neg1

    dummy = jnp.full((L,), DUMMY, jnp.int32)

    def pre_body(i, c):
        s = pl.ds(i * L, L)
        mj_v[s] = zero_v
        mo_v[s] = dummy
        us_v[s] = zero_v
        uo_v[s] = dummy
        return c

    lax.fori_loop(0, LIST // L, pre_body, 0)

    def scat_body(i, nbad):
        v = idx_v[pl.ds(i * L, L)]
        m = (v >= lo) & (v < lo + RANGE)
        t = jnp.where(m, v - lo, 0)
        j = i * L + iota
        plsc.store_scatter(tbl_v, [t], j, mask=m)
        g = plsc.load_gather(tbl_v, [t])
        flag = m & (g < j)
        fi = jnp.where(flag, 1, 0).astype(jnp.int32)
        pos = nbad + plsc.cumsum(fi) - 1
        posc = jnp.where(flag, pos, 0)
        packed = t * N_ADD + j
        plsc.store_scatter(bad_v, [posc], packed, mask=flag)
        return nbad + plsc.all_reduce_population_count(flag)

    nbad_v = lax.fori_loop(0, N_ADD // L, scat_body, zero_v, unroll=4)
    nbad = jnp.max(nbad_v)

    def fix_body(i, c):
        chunk = lax.shift_right_logical(i, 4)
        lane = lax.bitwise_and(i, L - 1)
        pk = bad_v[pl.ds(chunk * L, L)]
        vp = lax.shift_right_logical(pk, 14)
        vj = lax.bitwise_and(pk, N_ADD - 1)
        onemask = iota == lane
        g = plsc.load_gather(tbl_v, [vp])
        plsc.store_scatter(tbl_v, [vp], jnp.maximum(g, vj), mask=onemask)
        return c

    lax.fori_loop(0, nbad, fix_body, 0)

    def smp_body(i, offs):
        om, ou, ot = offs
        sv = smp_v[pl.ds(i * L, L)]
        m = (sv >= lo) & (sv < lo + RANGE)
        t = jnp.where(m, sv - lo, 0)
        g = plsc.load_gather(tbl_v, [t])
        mm = m & (g >= 0)
        mru = m & (g < 0)
        mt = mru & (sv >= TAILBASE)
        mu = mru & (sv < TAILBASE)
        spos = i * L + iota
        im = jnp.where(mm, 1, 0).astype(jnp.int32)
        iu = jnp.where(mu, 1, 0).astype(jnp.int32)
        it = jnp.where(mt, 1, 0).astype(jnp.int32)
        pm = om + plsc.cumsum(im) - 1
        pu = ou + plsc.cumsum(iu) - 1
        pt = (N_SAMPLE - 1) - (ot + plsc.cumsum(it) - 1)
        pmc = jnp.where(mm, pm, 0)
        puc = jnp.where(mu, pu, 0)
        ptc = jnp.where(mt, pt, 0)
        plsc.store_scatter(mj_v, [pmc], g, mask=mm)
        plsc.store_scatter(mo_v, [pmc], spos, mask=mm)
        plsc.store_scatter(us_v, [puc], sv, mask=mu)
        plsc.store_scatter(uo_v, [puc], spos, mask=mu)
        plsc.store_scatter(us_v, [ptc], sv, mask=mt)
        plsc.store_scatter(uo_v, [ptc], spos, mask=mt)
        return (om + plsc.all_reduce_population_count(mm),
                ou + plsc.all_reduce_population_count(mu),
                ot + plsc.all_reduce_population_count(mt))

    nm_v, nu_v, nt_v = lax.fori_loop(0, N_SAMPLE // L, smp_body,
                                     (zero_v, zero_v, zero_v), unroll=2)
    nm = jnp.max(nm_v)
    nu = jnp.max(nu_v)
    nt = jnp.max(nt_v)

    def assemble(b, lane_splat):
        bsel = jnp.full((L,), b, jnp.int32)
        g0 = plsc.load_gather(tb_v, [bsel, iota, lane_splat])
        g1 = plsc.load_gather(tb_v, [bsel, iota + L, lane_splat])
        rows_v[b, pl.ds(0, L)] = g0
        rows_v[b, pl.ds(L, L)] = g1

    def move_rows(src_list, dst_list, src_t, n):
        def group_body(g, carry):
            with jax.named_scope("g_extract"):
                sv16 = src_list[pl.ds(g * GRP, L)]
                svals, ofss = [], []
                for b in range(GRP):
                    s = jnp.max(jnp.where(iota == b, sv16, 0))
                    svals.append(s)
                    ofss.append(pl.multiple_of(
                        lax.shift_left(lax.shift_right_logical(s, 7), 7),
                        WIN))
            with jax.named_scope("g_fetch"):
                descs = [pltpu.async_copy(
                    src_t.at[:, pl.ds(ofss[b], WIN)], tb_v.at[b], sem)
                    for b in range(GRP)]
                dv = dst_list[pl.ds(g * GRP, L)]
                dvx = jnp.where(iota < GRP, dv, DUMMY)
                for d in descs:
                    d.wait()
            with jax.named_scope("g_asm"):
                for b in range(GRP):
                    lane = jnp.zeros((L,), jnp.int32) + lax.bitwise_and(
                        svals[b], WIN - 1)
                    assemble(b, lane)
            with jax.named_scope("g_scat"):
                pltpu.async_copy(rows_v, out_h.at[dvx], sem2).wait()
            return carry

        lax.fori_loop(0, (n + GRP - 1) // GRP, group_body, 0)

    with jax.named_scope("mv_matched"):
        move_rows(mj_v, mo_v, val_t, nm)
    with jax.named_scope("mv_unmatched"):
        move_rows(us_v, uo_v, mem_t, nu)

    pltpu.async_copy(tail_t, tb_v.at[0], sem).wait()

    def tail_body(i, carry):
        e = (N_SAMPLE - 1) - i
        ch = lax.shift_right_logical(e, 4)
        ln = lax.bitwise_and(e, L - 1)
        sv = us_v[pl.ds(ch * L, L)]
        dvv = uo_v[pl.ds(ch * L, L)]
        s_spl = _bcast(sv, ln)
        d_spl = _bcast(dvv, ln)
        lane = s_spl - TAILSTART
        bsel = jnp.zeros((L,), jnp.int32)
        g0 = plsc.load_gather(tb_v, [bsel, iota, lane])
        g1 = plsc.load_gather(tb_v, [bsel, iota + L, lane])
        rows_v[0, pl.ds(0, L)] = g0
        rows_v[0, pl.ds(L, L)] = g1
        dvx = jnp.where(iota == 0, d_spl, DUMMY)
        pltpu.async_copy(rows_v, out_h.at[dvx], sem2).wait()
        return carry

    lax.fori_loop(0, nt, tail_body, 0)


_sc_call_cache = []


def _get_sc_call():
    if not _sc_call_cache:
        _sc_call_cache.append(_build_sc_call())
    return _sc_call_cache[0]


def _build_sc_call():
    return pl.kernel(
        _body,
        out_type=jax.ShapeDtypeStruct((N_SAMPLE + 8, 128), jnp.float32),
        mesh=plsc.VectorSubcoreMesh(core_axis_name="c", subcore_axis_name="s"),
        compiler_params=pltpu.CompilerParams(needs_layout_passes=False),
        scratch_types=[
            pltpu.VMEM((N_ADD,), jnp.int32),
            pltpu.VMEM((N_SAMPLE,), jnp.int32),
            pltpu.VMEM((TBL,), jnp.int32),
            pltpu.VMEM((N_ADD,), jnp.int32),
            pltpu.VMEM((LIST,), jnp.int32),
            pltpu.VMEM((LIST,), jnp.int32),
            pltpu.VMEM((LIST,), jnp.int32),
            pltpu.VMEM((LIST,), jnp.int32),
            pltpu.VMEM((GRP, DIM, WIN), jnp.float32),
            pltpu.VMEM((L, 128), jnp.float32),
            pltpu.SemaphoreType.DMA,
            pltpu.SemaphoreType.DMA,
        ],
    )


def kernel(mem, idx, val, sample_idx):
    mem_t = mem.T
    tail_t = mem_t[:, TAILSTART:]
    out = _get_sc_call()(mem_t, idx, val.T, sample_idx, tail_t)
    return out[:N_SAMPLE, :DIM]


# speedup vs baseline: 1.0491x; 1.0482x over previous
import jax
import jax.numpy as jnp
from jax import lax
from jax.experimental import pallas as pl
from jax.experimental.pallas import tpu as pltpu
from jax.experimental.pallas import tpu_sc as plsc

CAP = 1_000_000
DIM = 32
N_ADD = 16384
N_SAMPLE = 4096
L = 16
NW = 32
RANGE = CAP // NW
TBL = RANGE + (-RANGE) % L
DUMMY = N_SAMPLE
WIN = 128
GRP = 8
SCH = 512
LIST = N_SAMPLE + L
TAILBASE = (CAP // WIN) * WIN
TAILSTART = CAP - WIN


def _bcast(x, r):
    return x.at[jnp.full((L,), r, jnp.int32)].get(mode="promise_in_bounds")


def _body(mem_t, idx_h, val_t, smp_h, tail_t, out_h,
          idx_v, smp_v, tbl_v, bad_v, mj_v, mo_v, us_v, uo_v,
          tb_v, rows_v, sem, sem2):
    cid = lax.axis_index("c")
    sid = lax.axis_index("s")
    wid = sid * 2 + cid
    lo = wid * RANGE
    iota = lax.iota(jnp.int32, L)
    zero_v = jnp.zeros((L,), jnp.int32)

    pltpu.sync_copy(idx_h, idx_v)
    pltpu.sync_copy(smp_h, smp_v)

    neg1 = jnp.full((L,), -1, jnp.int32)

    def init_body(i, c):
        for u in range(8):
            tbl_v[pl.ds(i * (8 * L) + u * L, L)] = neg1
        return c

    lax.fori_loop(0, TBL // (8 * L), init_body, 0)
    for u in range((TBL % (8 * L)) // L):
        tbl_v[pl.ds(TBL - (TBL % (8 * L)) + u * L, L)] = neg1

    dummy = jnp.full((L,), DUMMY, jnp.int32)

    def pre_body(i, c):
        s = pl.ds(i * L, L)
        mj_v[s] = zero_v
        mo_v[s] = dummy
        us_v[s] = zero_v
        uo_v[s] = dummy
        return c

    lax.fori_loop(0, LIST // L, pre_body, 0)

    def scat_body(i, nbad):
        v = idx_v[pl.ds(i * L, L)]
        m = (v >= lo) & (v < lo + RANGE)
        t = jnp.where(m, v - lo, 0)
        j = i * L + iota
        plsc.store_scatter(tbl_v, [t], j, mask=m)
        g = plsc.load_gather(tbl_v, [t])
        flag = m & (g < j)
        fi = jnp.where(flag, 1, 0).astype(jnp.int32)
        pos = nbad + plsc.cumsum(fi) - 1
        posc = jnp.where(flag, pos, 0)
        packed = t * N_ADD + j
        plsc.store_scatter(bad_v, [posc], packed, mask=flag)
        return nbad + plsc.all_reduce_population_count(flag)

    nbad_v = lax.fori_loop(0, N_ADD // L, scat_body, zero_v, unroll=4)
    nbad = jnp.max(nbad_v)

    def fix_body(i, c):
        chunk = lax.shift_right_logical(i, 4)
        lane = lax.bitwise_and(i, L - 1)
        pk = bad_v[pl.ds(chunk * L, L)]
        vp = lax.shift_right_logical(pk, 14)
        vj = lax.bitwise_and(pk, N_ADD - 1)
        onemask = iota == lane
        g = plsc.load_gather(tbl_v, [vp])
        plsc.store_scatter(tbl_v, [vp], jnp.maximum(g, vj), mask=onemask)
        return c

    lax.fori_loop(0, nbad, fix_body, 0)

    def smp_body(i, offs):
        om, ou, ot = offs
        sv = smp_v[pl.ds(i * L, L)]
        m = (sv >= lo) & (sv < lo + RANGE)
        t = jnp.where(m, sv - lo, 0)
        g = plsc.load_gather(tbl_v, [t])
        mm = m & (g >= 0)
        mru = m & (g < 0)
        mt = mru & (sv >= TAILBASE)
        mu = mru & (sv < TAILBASE)
        spos = i * L + iota
        im = jnp.where(mm, 1, 0).astype(jnp.int32)
        iu = jnp.where(mu, 1, 0).astype(jnp.int32)
        it = jnp.where(mt, 1, 0).astype(jnp.int32)
        pm = om + plsc.cumsum(im) - 1
        pu = ou + plsc.cumsum(iu) - 1
        pt = (N_SAMPLE - 1) - (ot + plsc.cumsum(it) - 1)
        pmc = jnp.where(mm, pm, 0)
        puc = jnp.where(mu, pu, 0)
        ptc = jnp.where(mt, pt, 0)
        plsc.store_scatter(mj_v, [pmc], g, mask=mm)
        plsc.store_scatter(mo_v, [pmc], spos, mask=mm)
        plsc.store_scatter(us_v, [puc], sv, mask=mu)
        plsc.store_scatter(uo_v, [puc], spos, mask=mu)
        plsc.store_scatter(us_v, [ptc], sv, mask=mt)
        plsc.store_scatter(uo_v, [ptc], spos, mask=mt)
        return (om + plsc.all_reduce_population_count(mm),
                ou + plsc.all_reduce_population_count(mu),
                ot + plsc.all_reduce_population_count(mt))

    nm_v, nu_v, nt_v = lax.fori_loop(0, N_SAMPLE // L, smp_body,
                                     (zero_v, zero_v, zero_v), unroll=2)
    nm = jnp.max(nm_v)
    nu = jnp.max(nu_v)
    nt = jnp.max(nt_v)

    dummy16 = jnp.full((L,), DUMMY, jnp.int32)

    def assemble(rb, b, lane_splat):
        bsel = jnp.full((L,), b, jnp.int32)
        g0 = plsc.load_gather(tb_v, [bsel, iota, lane_splat])
        g1 = plsc.load_gather(tb_v, [bsel, iota + L, lane_splat])
        rows_v[rb, b, pl.ds(0, L)] = g0
        rows_v[rb, b, pl.ds(L, L)] = g1

    def move_rows(src_list, dst_list, src_t, n):
        def group_body(g, carry):
            with jax.named_scope("g_extract"):
                sv16 = src_list[pl.ds(g * GRP, L)]
                svals, ofss = [], []
                for b in range(GRP):
                    s = jnp.max(jnp.where(iota == b, sv16, 0))
                    svals.append(s)
                    ofss.append(pl.multiple_of(
                        lax.shift_left(lax.shift_right_logical(s, 7), 7),
                        WIN))
            with jax.named_scope("g_fetch"):
                descs = [pltpu.async_copy(
                    src_t.at[:, pl.ds(ofss[b], WIN)], tb_v.at[b], sem)
                    for b in range(GRP)]
                dv = dst_list[pl.ds(g * GRP, L)]
                dvx = jnp.where(iota < GRP, dv, DUMMY)
                for d in descs:
                    d.wait()
            rb = lax.bitwise_and(g, 3)
            with jax.named_scope("g_drain"):
                @pl.when(g >= 4)
                def _drain():
                    pltpu.make_async_copy(out_h.at[dummy16], rows_v.at[rb],
                                          sem2).wait()
            with jax.named_scope("g_asm"):
                for b in range(GRP):
                    lane = jnp.zeros((L,), jnp.int32) + lax.bitwise_and(
                        svals[b], WIN - 1)
                    assemble(rb, b, lane)
            with jax.named_scope("g_scat"):
                pltpu.async_copy(rows_v.at[rb], out_h.at[dvx], sem2)
            return carry

        ngrp = (n + GRP - 1) // GRP
        lax.fori_loop(0, ngrp, group_body, 0)

        def drain_body(i, carry):
            pltpu.make_async_copy(out_h.at[dummy16], rows_v.at[0],
                                  sem2).wait()
            return carry

        lax.fori_loop(0, jnp.minimum(ngrp, 4), drain_body, 0)

    with jax.named_scope("mv_matched"):
        move_rows(mj_v, mo_v, val_t, nm)
    with jax.named_scope("mv_unmatched"):
        move_rows(us_v, uo_v, mem_t, nu)

    pltpu.async_copy(tail_t, tb_v.at[0], sem).wait()

    def tail_body(i, carry):
        e = (N_SAMPLE - 1) - i
        ch = lax.shift_right_logical(e, 4)
        ln = lax.bitwise_and(e, L - 1)
        sv = us_v[pl.ds(ch * L, L)]
        dvv = uo_v[pl.ds(ch * L, L)]
        s_spl = _bcast(sv, ln)
        d_spl = _bcast(dvv, ln)
        lane = s_spl - TAILSTART
        bsel = jnp.zeros((L,), jnp.int32)
        g0 = plsc.load_gather(tb_v, [bsel, iota, lane])
        g1 = plsc.load_gather(tb_v, [bsel, iota + L, lane])
        rows_v[0, 0, pl.ds(0, L)] = g0
        rows_v[0, 0, pl.ds(L, L)] = g1
        dvx = jnp.where(iota == 0, d_spl, DUMMY)
        pltpu.async_copy(rows_v.at[0], out_h.at[dvx], sem2).wait()
        return carry

    lax.fori_loop(0, nt, tail_body, 0)


_sc_call_cache = []


def _get_sc_call():
    if not _sc_call_cache:
        _sc_call_cache.append(_build_sc_call())
    return _sc_call_cache[0]


def _build_sc_call():
    return pl.kernel(
        _body,
        out_type=jax.ShapeDtypeStruct((N_SAMPLE + 8, 128), jnp.float32),
        mesh=plsc.VectorSubcoreMesh(core_axis_name="c", subcore_axis_name="s"),
        compiler_params=pltpu.CompilerParams(needs_layout_passes=False),
        scratch_types=[
            pltpu.VMEM((N_ADD,), jnp.int32),
            pltpu.VMEM((N_SAMPLE,), jnp.int32),
            pltpu.VMEM((TBL,), jnp.int32),
            pltpu.VMEM((N_ADD,), jnp.int32),
            pltpu.VMEM((LIST,), jnp.int32),
            pltpu.VMEM((LIST,), jnp.int32),
            pltpu.VMEM((LIST,), jnp.int32),
            pltpu.VMEM((LIST,), jnp.int32),
            pltpu.VMEM((GRP, DIM, WIN), jnp.float32),
            pltpu.VMEM((4, L, 128), jnp.float32),
            pltpu.SemaphoreType.DMA,
            pltpu.SemaphoreType.DMA,
        ],
    )


def kernel(mem, idx, val, sample_idx):
    mem_t = mem.T
    tail_t = mem_t[:, TAILSTART:]
    out = _get_sc_call()(mem_t, idx, val.T, sample_idx, tail_t)
    return out[:N_SAMPLE, :DIM]


# speedup vs baseline: 2.1605x; 2.0593x over previous
import jax
import jax.numpy as jnp
from jax import lax
from jax.experimental import pallas as pl
from jax.experimental.pallas import tpu as pltpu
from jax.experimental.pallas import tpu_sc as plsc

CAP = 1_000_000
DIM = 32
N_ADD = 16384
N_SAMPLE = 4096
L = 16
NW = 32
RANGE = CAP // NW
TBL = RANGE + (-RANGE) % L
DUMMY = N_SAMPLE
WIN = 128
GRP = 8
SCH = 512
LIST = N_SAMPLE + L
TAILBASE = (CAP // WIN) * WIN
TAILSTART = CAP - WIN


def _bcast(x, r):
    return x.at[jnp.full((L,), r, jnp.int32)].get(mode="promise_in_bounds")


def _body(mem_t, idx_h, val_t, smp_h, tail_t, out_h,
          idx_v, smp_v, tbl_v, bad_v, mj_v, mo_v, us_v, uo_v,
          tb_v, rows_v, sem, sem2):
    cid = lax.axis_index("c")
    sid = lax.axis_index("s")
    wid = sid * 2 + cid
    lo = wid * RANGE
    iota = lax.iota(jnp.int32, L)
    zero_v = jnp.zeros((L,), jnp.int32)

    pltpu.sync_copy(idx_h, idx_v)
    pltpu.sync_copy(smp_h, smp_v)

    neg1 = jnp.full((L,), -1, jnp.int32)

    def init_body(i, c):
        for u in range(8):
            tbl_v[pl.ds(i * (8 * L) + u * L, L)] = neg1
        return c

    lax.fori_loop(0, TBL // (8 * L), init_body, 0)
    for u in range((TBL % (8 * L)) // L):
        tbl_v[pl.ds(TBL - (TBL % (8 * L)) + u * L, L)] = neg1

    dummy = jnp.full((L,), DUMMY, jnp.int32) + iota

    def pre_body(i, c):
        s = pl.ds(i * L, L)
        mj_v[s] = zero_v
        mo_v[s] = dummy
        us_v[s] = zero_v
        uo_v[s] = dummy
        return c

    lax.fori_loop(0, LIST // L, pre_body, 0)

    def scat_body(i, nbad):
        v = idx_v[pl.ds(i * L, L)]
        m = (v >= lo) & (v < lo + RANGE)
        t = jnp.where(m, v - lo, 0)
        j = i * L + iota
        plsc.store_scatter(tbl_v, [t], j, mask=m)
        g = plsc.load_gather(tbl_v, [t])
        flag = m & (g < j)
        fi = jnp.where(flag, 1, 0).astype(jnp.int32)
        pos = nbad + plsc.cumsum(fi) - 1
        posc = jnp.where(flag, pos, 0)
        packed = t * N_ADD + j
        plsc.store_scatter(bad_v, [posc], packed, mask=flag)
        return nbad + plsc.all_reduce_population_count(flag)

    nbad_v = lax.fori_loop(0, N_ADD // L, scat_body, zero_v, unroll=4)
    nbad = jnp.max(nbad_v)

    def fix_body(i, c):
        chunk = lax.shift_right_logical(i, 4)
        lane = lax.bitwise_and(i, L - 1)
        pk = bad_v[pl.ds(chunk * L, L)]
        vp = lax.shift_right_logical(pk, 14)
        vj = lax.bitwise_and(pk, N_ADD - 1)
        onemask = iota == lane
        g = plsc.load_gather(tbl_v, [vp])
        plsc.store_scatter(tbl_v, [vp], jnp.maximum(g, vj), mask=onemask)
        return c

    lax.fori_loop(0, nbad, fix_body, 0)

    def smp_body(i, offs):
        om, ou, ot = offs
        sv = smp_v[pl.ds(i * L, L)]
        m = (sv >= lo) & (sv < lo + RANGE)
        t = jnp.where(m, sv - lo, 0)
        g = plsc.load_gather(tbl_v, [t])
        mm = m & (g >= 0)
        mru = m & (g < 0)
        mt = mru & (sv >= TAILBASE)
        mu = mru & (sv < TAILBASE)
        spos = i * L + iota
        im = jnp.where(mm, 1, 0).astype(jnp.int32)
        iu = jnp.where(mu, 1, 0).astype(jnp.int32)
        it = jnp.where(mt, 1, 0).astype(jnp.int32)
        pm = om + plsc.cumsum(im) - 1
        pu = ou + plsc.cumsum(iu) - 1
        pt = (N_SAMPLE - 1) - (ot + plsc.cumsum(it) - 1)
        pmc = jnp.where(mm, pm, 0)
        puc = jnp.where(mu, pu, 0)
        ptc = jnp.where(mt, pt, 0)
        plsc.store_scatter(mj_v, [pmc], g, mask=mm)
        plsc.store_scatter(mo_v, [pmc], spos, mask=mm)
        plsc.store_scatter(us_v, [puc], sv, mask=mu)
        plsc.store_scatter(uo_v, [puc], spos, mask=mu)
        plsc.store_scatter(us_v, [ptc], sv, mask=mt)
        plsc.store_scatter(uo_v, [ptc], spos, mask=mt)
        return (om + plsc.all_reduce_population_count(mm),
                ou + plsc.all_reduce_population_count(mu),
                ot + plsc.all_reduce_population_count(mt))

    nm_v, nu_v, nt_v = lax.fori_loop(0, N_SAMPLE // L, smp_body,
                                     (zero_v, zero_v, zero_v), unroll=2)
    nm = jnp.max(nm_v)
    nu = jnp.max(nu_v)
    nt = jnp.max(nt_v)

    dummy16 = jnp.full((L,), DUMMY, jnp.int32)

    def assemble(rb, b, lane_splat):
        bsel = jnp.full((L,), b, jnp.int32)
        g0 = plsc.load_gather(tb_v, [bsel, iota, lane_splat])
        g1 = plsc.load_gather(tb_v, [bsel, iota + L, lane_splat])
        rows_v[rb, b, pl.ds(0, L)] = g0
        rows_v[rb, b, pl.ds(L, L)] = g1

    def move_rows(src_list, dst_list, src_t, n):
        def group_body(g, carry):
            with jax.named_scope("g_extract"):
                sv16 = src_list[pl.ds(g * GRP, L)]
                svals, ofss = [], []
                for b in range(GRP):
                    s = jnp.max(jnp.where(iota == b, sv16, 0))
                    svals.append(s)
                    ofss.append(pl.multiple_of(
                        lax.shift_left(lax.shift_right_logical(s, 7), 7),
                        WIN))
            with jax.named_scope("g_fetch"):
                descs = [pltpu.async_copy(
                    src_t.at[:, pl.ds(ofss[b], WIN)], tb_v.at[b], sem)
                    for b in range(GRP)]
                dv = dst_list[pl.ds(g * GRP, L)]
                dvx = jnp.where(iota < GRP, dv, DUMMY + iota)
                for d in descs:
                    d.wait()
            rb = lax.bitwise_and(g, 3)
            with jax.named_scope("g_drain"):
                @pl.when(g >= 4)
                def _drain():
                    pltpu.make_async_copy(out_h.at[dummy16], rows_v.at[rb],
                                          sem2).wait()
            with jax.named_scope("g_asm"):
                for b in range(GRP):
                    lane = jnp.zeros((L,), jnp.int32) + lax.bitwise_and(
                        svals[b], WIN - 1)
                    assemble(rb, b, lane)
            with jax.named_scope("g_scat"):
                pltpu.async_copy(rows_v.at[rb], out_h.at[dvx], sem2)
            return carry

        ngrp = (n + GRP - 1) // GRP
        lax.fori_loop(0, ngrp, group_body, 0)

        def drain_body(i, carry):
            pltpu.make_async_copy(out_h.at[dummy16], rows_v.at[0],
                                  sem2).wait()
            return carry

        lax.fori_loop(0, jnp.minimum(ngrp, 4), drain_body, 0)

    with jax.named_scope("mv_matched"):
        move_rows(mj_v, mo_v, val_t, nm)
    with jax.named_scope("mv_unmatched"):
        move_rows(us_v, uo_v, mem_t, nu)

    pltpu.async_copy(tail_t, tb_v.at[0], sem).wait()

    def tail_body(i, carry):
        e = (N_SAMPLE - 1) - i
        ch = lax.shift_right_logical(e, 4)
        ln = lax.bitwise_and(e, L - 1)
        sv = us_v[pl.ds(ch * L, L)]
        dvv = uo_v[pl.ds(ch * L, L)]
        s_spl = _bcast(sv, ln)
        d_spl = _bcast(dvv, ln)
        lane = s_spl - TAILSTART
        bsel = jnp.zeros((L,), jnp.int32)
        g0 = plsc.load_gather(tb_v, [bsel, iota, lane])
        g1 = plsc.load_gather(tb_v, [bsel, iota + L, lane])
        rows_v[0, 0, pl.ds(0, L)] = g0
        rows_v[0, 0, pl.ds(L, L)] = g1
        dvx = jnp.where(iota == 0, d_spl, DUMMY + iota)
        pltpu.async_copy(rows_v.at[0], out_h.at[dvx], sem2).wait()
        return carry

    lax.fori_loop(0, nt, tail_body, 0)


_sc_call_cache = []


def _get_sc_call():
    if not _sc_call_cache:
        _sc_call_cache.append(_build_sc_call())
    return _sc_call_cache[0]


def _build_sc_call():
    return pl.kernel(
        _body,
        out_type=jax.ShapeDtypeStruct((N_SAMPLE + 2 * L, 128), jnp.float32),
        mesh=plsc.VectorSubcoreMesh(core_axis_name="c", subcore_axis_name="s"),
        compiler_params=pltpu.CompilerParams(needs_layout_passes=False),
        scratch_types=[
            pltpu.VMEM((N_ADD,), jnp.int32),
            pltpu.VMEM((N_SAMPLE,), jnp.int32),
            pltpu.VMEM((TBL,), jnp.int32),
            pltpu.VMEM((N_ADD,), jnp.int32),
            pltpu.VMEM((LIST,), jnp.int32),
            pltpu.VMEM((LIST,), jnp.int32),
            pltpu.VMEM((LIST,), jnp.int32),
            pltpu.VMEM((LIST,), jnp.int32),
            pltpu.VMEM((GRP, DIM, WIN), jnp.float32),
            pltpu.VMEM((4, L, 128), jnp.float32),
            pltpu.SemaphoreType.DMA,
            pltpu.SemaphoreType.DMA,
        ],
    )


def kernel(mem, idx, val, sample_idx):
    mem_t = mem.T
    tail_t = mem_t[:, TAILSTART:]
    out = _get_sc_call()(mem_t, idx, val.T, sample_idx, tail_t)
    return out[:N_SAMPLE, :DIM]


# speedup vs baseline: 2.1682x; 1.0035x over previous
import jax
import jax.numpy as jnp
from jax import lax
from jax.experimental import pallas as pl
from jax.experimental.pallas import tpu as pltpu
from jax.experimental.pallas import tpu_sc as plsc

CAP = 1_000_000
DIM = 32
N_ADD = 16384
N_SAMPLE = 4096
L = 16
NW = 32
RANGE = CAP // NW
TBL = RANGE + (-RANGE) % L
DUMMY = N_SAMPLE
WIN = 128
GRP = 8
SCH = 512
LIST = N_SAMPLE + L
TAILBASE = (CAP // WIN) * WIN
TAILSTART = CAP - WIN


def _bcast(x, r):
    return x.at[jnp.full((L,), r, jnp.int32)].get(mode="promise_in_bounds")


def _body(mem_t, idx_h, val_t, smp_h, tail_t, out_h,
          idx_v, smp_v, tbl_v, bad_v, mj_v, mo_v, us_v, uo_v,
          tb_v, rows_v, sem, sem2):
    cid = lax.axis_index("c")
    sid = lax.axis_index("s")
    wid = sid * 2 + cid
    lo = wid * RANGE
    iota = lax.iota(jnp.int32, L)
    zero_v = jnp.zeros((L,), jnp.int32)

    pltpu.sync_copy(idx_h, idx_v)
    pltpu.sync_copy(smp_h, smp_v)

    neg1 = jnp.full((L,), -1, jnp.int32)

    def init_body(i, c):
        for u in range(8):
            tbl_v[pl.ds(i * (8 * L) + u * L, L)] = neg1
        return c

    lax.fori_loop(0, TBL // (8 * L), init_body, 0)
    for u in range((TBL % (8 * L)) // L):
        tbl_v[pl.ds(TBL - (TBL % (8 * L)) + u * L, L)] = neg1

    dummy = jnp.full((L,), DUMMY, jnp.int32) + iota

    def pre_body(i, c):
        s = pl.ds(i * L, L)
        mj_v[s] = zero_v
        mo_v[s] = dummy
        us_v[s] = zero_v
        uo_v[s] = dummy
        return c

    lax.fori_loop(0, LIST // L, pre_body, 0)

    def scat_body(i, nbad):
        v = idx_v[pl.ds(i * L, L)]
        m = (v >= lo) & (v < lo + RANGE)
        t = jnp.where(m, v - lo, 0)
        j = i * L + iota
        plsc.store_scatter(tbl_v, [t], j, mask=m)
        g = plsc.load_gather(tbl_v, [t])
        flag = m & (g < j)
        packed = t * N_ADD + j
        plsc.store_compressed(bad_v.at[pl.ds(nbad, L)], packed, mask=flag)
        return nbad + plsc.all_reduce_population_count(flag)[0]

    nbad = lax.fori_loop(0, N_ADD // L, scat_body, jnp.int32(0), unroll=4)

    def fix_body(i, c):
        chunk = lax.shift_right_logical(i, 4)
        lane = lax.bitwise_and(i, L - 1)
        pk = bad_v[pl.ds(chunk * L, L)]
        vp = lax.shift_right_logical(pk, 14)
        vj = lax.bitwise_and(pk, N_ADD - 1)
        onemask = iota == lane
        g = plsc.load_gather(tbl_v, [vp])
        plsc.store_scatter(tbl_v, [vp], jnp.maximum(g, vj), mask=onemask)
        return c

    lax.fori_loop(0, nbad, fix_body, 0)

    def smp_body(i, offs):
        om, ou, ot = offs
        sv = smp_v[pl.ds(i * L, L)]
        m = (sv >= lo) & (sv < lo + RANGE)
        t = jnp.where(m, sv - lo, 0)
        g = plsc.load_gather(tbl_v, [t])
        mm = m & (g >= 0)
        mru = m & (g < 0)
        mt = mru & (sv >= TAILBASE)
        mu = mru & (sv < TAILBASE)
        spos = i * L + iota
        plsc.store_compressed(mj_v.at[pl.ds(om, L)], g, mask=mm)
        plsc.store_compressed(mo_v.at[pl.ds(om, L)], spos, mask=mm)
        plsc.store_compressed(us_v.at[pl.ds(ou, L)], sv, mask=mu)
        plsc.store_compressed(uo_v.at[pl.ds(ou, L)], spos, mask=mu)
        nt16 = plsc.all_reduce_population_count(mt)

        @pl.when(nt16[0] > 0)
        def _tail_divert():
            it = jnp.where(mt, 1, 0).astype(jnp.int32)
            pt = (N_SAMPLE - 1) - (ot + plsc.cumsum(it) - 1)
            ptc = jnp.where(mt, pt, 0)
            plsc.store_scatter(us_v, [ptc], sv, mask=mt)
            plsc.store_scatter(uo_v, [ptc], spos, mask=mt)

        return (om + plsc.all_reduce_population_count(mm)[0],
                ou + plsc.all_reduce_population_count(mu)[0],
                ot + nt16[0])

    nm, nu, nt = lax.fori_loop(
        0, N_SAMPLE // L, smp_body,
        (jnp.int32(0), jnp.int32(0), jnp.int32(0)), unroll=2)

    dummy16 = jnp.full((L,), DUMMY, jnp.int32)

    def assemble(rb, b, lane_splat):
        bsel = jnp.full((L,), b, jnp.int32)
        g0 = plsc.load_gather(tb_v, [bsel, iota, lane_splat])
        g1 = plsc.load_gather(tb_v, [bsel, iota + L, lane_splat])
        rows_v[rb, b, pl.ds(0, L)] = g0
        rows_v[rb, b, pl.ds(L, L)] = g1

    def move_rows(src_list, dst_list, src_t, n):
        def group_body(g, carry):
            with jax.named_scope("g_extract"):
                sv16 = src_list[pl.ds(g * GRP, L)]
                svals, ofss = [], []
                for b in range(GRP):
                    s = jnp.max(jnp.where(iota == b, sv16, 0))
                    svals.append(s)
                    ofss.append(pl.multiple_of(
                        lax.shift_left(lax.shift_right_logical(s, 7), 7),
                        WIN))
            with jax.named_scope("g_fetch"):
                descs = [pltpu.async_copy(
                    src_t.at[:, pl.ds(ofss[b], WIN)], tb_v.at[b], sem)
                    for b in range(GRP)]
                dv = dst_list[pl.ds(g * GRP, L)]
                dvx = jnp.where(iota < GRP, dv, DUMMY + iota)
                for d in descs:
                    d.wait()
            rb = lax.bitwise_and(g, 3)
            with jax.named_scope("g_drain"):
                @pl.when(g >= 4)
                def _drain():
                    pltpu.make_async_copy(out_h.at[dummy16], rows_v.at[rb],
                                          sem2).wait()
            with jax.named_scope("g_asm"):
                for b in range(GRP):
                    lane = jnp.zeros((L,), jnp.int32) + lax.bitwise_and(
                        svals[b], WIN - 1)
                    assemble(rb, b, lane)
            with jax.named_scope("g_scat"):
                pltpu.async_copy(rows_v.at[rb], out_h.at[dvx], sem2)
            return carry

        ngrp = (n + GRP - 1) // GRP
        lax.fori_loop(0, ngrp, group_body, 0)

        def drain_body(i, carry):
            pltpu.make_async_copy(out_h.at[dummy16], rows_v.at[0],
                                  sem2).wait()
            return carry

        lax.fori_loop(0, jnp.minimum(ngrp, 4), drain_body, 0)

    with jax.named_scope("mv_matched"):
        move_rows(mj_v, mo_v, val_t, nm)
    with jax.named_scope("mv_unmatched"):
        move_rows(us_v, uo_v, mem_t, nu)

    pltpu.async_copy(tail_t, tb_v.at[0], sem).wait()

    def tail_body(i, carry):
        e = (N_SAMPLE - 1) - i
        ch = lax.shift_right_logical(e, 4)
        ln = lax.bitwise_and(e, L - 1)
        sv = us_v[pl.ds(ch * L, L)]
        dvv = uo_v[pl.ds(ch * L, L)]
        s_spl = _bcast(sv, ln)
        d_spl = _bcast(dvv, ln)
        lane = s_spl - TAILSTART
        bsel = jnp.zeros((L,), jnp.int32)
        g0 = plsc.load_gather(tb_v, [bsel, iota, lane])
        g1 = plsc.load_gather(tb_v, [bsel, iota + L, lane])
        rows_v[0, 0, pl.ds(0, L)] = g0
        rows_v[0, 0, pl.ds(L, L)] = g1
        dvx = jnp.where(iota == 0, d_spl, DUMMY + iota)
        pltpu.async_copy(rows_v.at[0], out_h.at[dvx], sem2).wait()
        return carry

    lax.fori_loop(0, nt, tail_body, 0)


_sc_call_cache = []


def _get_sc_call():
    if not _sc_call_cache:
        _sc_call_cache.append(_build_sc_call())
    return _sc_call_cache[0]


def _build_sc_call():
    return pl.kernel(
        _body,
        out_type=jax.ShapeDtypeStruct((N_SAMPLE + 2 * L, 128), jnp.float32),
        mesh=plsc.VectorSubcoreMesh(core_axis_name="c", subcore_axis_name="s"),
        compiler_params=pltpu.CompilerParams(needs_layout_passes=False),
        scratch_types=[
            pltpu.VMEM((N_ADD,), jnp.int32),
            pltpu.VMEM((N_SAMPLE,), jnp.int32),
            pltpu.VMEM((TBL,), jnp.int32),
            pltpu.VMEM((N_ADD + L,), jnp.int32),
            pltpu.VMEM((LIST,), jnp.int32),
            pltpu.VMEM((LIST,), jnp.int32),
            pltpu.VMEM((LIST,), jnp.int32),
            pltpu.VMEM((LIST,), jnp.int32),
            pltpu.VMEM((GRP, DIM, WIN), jnp.float32),
            pltpu.VMEM((4, L, 128), jnp.float32),
            pltpu.SemaphoreType.DMA,
            pltpu.SemaphoreType.DMA,
        ],
    )


def kernel(mem, idx, val, sample_idx):
    mem_t = mem.T
    tail_t = mem_t[:, TAILSTART:]
    out = _get_sc_call()(mem_t, idx, val.T, sample_idx, tail_t)
    return out[:N_SAMPLE, :DIM]


# speedup vs baseline: 2.1746x; 1.0029x over previous
import jax
import jax.numpy as jnp
from jax import lax
from jax.experimental import pallas as pl
from jax.experimental.pallas import tpu as pltpu
from jax.experimental.pallas import tpu_sc as plsc

CAP = 1_000_000
DIM = 32
N_ADD = 16384
N_SAMPLE = 4096
L = 16
NW = 32
RANGE = CAP // NW
TBL = RANGE + (-RANGE) % L
DUMMY = N_SAMPLE
WIN = 128
GRP = 8
SCH = 512
LIST = N_SAMPLE + L
TAILBASE = (CAP // WIN) * WIN
TAILSTART = CAP - WIN


def _bcast(x, r):
    return x.at[jnp.full((L,), r, jnp.int32)].get(mode="promise_in_bounds")


def _body(mem_t, idx_h, val_t, smp_h, tail_t, out_h,
          idx_v, smp_v, tbl_v, bad_v, mj_v, mo_v, us_v, uo_v,
          tb_v, rows_v, sem, sem2):
    cid = lax.axis_index("c")
    sid = lax.axis_index("s")
    wid = sid * 2 + cid
    lo = wid * RANGE
    iota = lax.iota(jnp.int32, L)
    zero_v = jnp.zeros((L,), jnp.int32)

    pltpu.sync_copy(idx_h, idx_v)
    pltpu.sync_copy(smp_h, smp_v)

    neg1 = jnp.full((L,), -1, jnp.int32)

    def init_body(i, c):
        for u in range(8):
            tbl_v[pl.ds(i * (8 * L) + u * L, L)] = neg1
        return c

    lax.fori_loop(0, TBL // (8 * L), init_body, 0)
    for u in range((TBL % (8 * L)) // L):
        tbl_v[pl.ds(TBL - (TBL % (8 * L)) + u * L, L)] = neg1

    dummy = jnp.full((L,), DUMMY, jnp.int32) + iota

    def pre_body(i, c):
        s = pl.ds(i * L, L)
        mj_v[s] = zero_v
        mo_v[s] = dummy
        us_v[s] = zero_v
        uo_v[s] = dummy
        return c

    lax.fori_loop(0, LIST // L, pre_body, 0)

    def scat_body(i, nbad):
        v = idx_v[pl.ds(i * L, L)]
        m = (v >= lo) & (v < lo + RANGE)
        t = jnp.where(m, v - lo, 0)
        j = i * L + iota
        plsc.store_scatter(tbl_v, [t], j, mask=m)
        g = plsc.load_gather(tbl_v, [t])
        flag = m & (g < j)
        packed = t * N_ADD + j
        plsc.store_compressed(bad_v.at[pl.ds(nbad, L)], packed, mask=flag)
        return nbad + plsc.all_reduce_population_count(flag)[0]

    nbad = lax.fori_loop(0, N_ADD // L, scat_body, jnp.int32(0), unroll=4)

    def fix_body(i, c):
        chunk = lax.shift_right_logical(i, 4)
        lane = lax.bitwise_and(i, L - 1)
        pk = bad_v[pl.ds(chunk * L, L)]
        vp = lax.shift_right_logical(pk, 14)
        vj = lax.bitwise_and(pk, N_ADD - 1)
        onemask = iota == lane
        g = plsc.load_gather(tbl_v, [vp])
        plsc.store_scatter(tbl_v, [vp], jnp.maximum(g, vj), mask=onemask)
        return c

    lax.fori_loop(0, nbad, fix_body, 0)

    def smp_body(i, offs):
        om, ou, ot = offs
        sv = smp_v[pl.ds(i * L, L)]
        m = (sv >= lo) & (sv < lo + RANGE)
        t = jnp.where(m, sv - lo, 0)
        g = plsc.load_gather(tbl_v, [t])
        mm = m & (g >= 0)
        mru = m & (g < 0)
        mt = mru & (sv >= TAILBASE)
        mu = mru & (sv < TAILBASE)
        spos = i * L + iota
        plsc.store_compressed(mj_v.at[pl.ds(om, L)], g, mask=mm)
        plsc.store_compressed(mo_v.at[pl.ds(om, L)], spos, mask=mm)
        plsc.store_compressed(us_v.at[pl.ds(ou, L)], sv, mask=mu)
        plsc.store_compressed(uo_v.at[pl.ds(ou, L)], spos, mask=mu)
        nt16 = plsc.all_reduce_population_count(mt)

        @pl.when(nt16[0] > 0)
        def _tail_divert():
            it = jnp.where(mt, 1, 0).astype(jnp.int32)
            pt = (N_SAMPLE - 1) - (ot + plsc.cumsum(it) - 1)
            ptc = jnp.where(mt, pt, 0)
            plsc.store_scatter(us_v, [ptc], sv, mask=mt)
            plsc.store_scatter(uo_v, [ptc], spos, mask=mt)

        return (om + plsc.all_reduce_population_count(mm)[0],
                ou + plsc.all_reduce_population_count(mu)[0],
                ot + nt16[0])

    nm, nu, nt = lax.fori_loop(
        0, N_SAMPLE // L, smp_body,
        (jnp.int32(0), jnp.int32(0), jnp.int32(0)), unroll=2)

    dummy16 = jnp.full((L,), DUMMY, jnp.int32)

    def assemble(rb, b, lane_splat):
        bsel = jnp.full((L,), b, jnp.int32)
        g0 = plsc.load_gather(tb_v, [bsel, iota, lane_splat])
        g1 = plsc.load_gather(tb_v, [bsel, iota + L, lane_splat])
        rows_v[rb, b, pl.ds(0, L)] = g0
        rows_v[rb, b, pl.ds(L, L)] = g1

    def move_rows(src_list, dst_list, src_t, n):
        def group_body(g, carry):
            sv16 = src_list[pl.ds(g * GRP, L)]
            svals, ofss = [], []
            for b in range(GRP):
                s = jnp.max(jnp.where(iota == b, sv16, 0))
                svals.append(s)
                ofss.append(pl.multiple_of(
                    lax.shift_left(lax.shift_right_logical(s, 7), 7), WIN))
            descs = [pltpu.async_copy(
                src_t.at[:, pl.ds(ofss[b], WIN)], tb_v.at[b], sem)
                for b in range(GRP)]
            dv = dst_list[pl.ds(g * GRP, L)]
            dvx = jnp.where(iota < GRP, dv, DUMMY + iota)
            for d in descs:
                d.wait()
            rb = lax.bitwise_and(g, 3)

            @pl.when(g >= 4)
            def _drain():
                pltpu.make_async_copy(out_h.at[dummy16], rows_v.at[rb],
                                      sem2).wait()

            for b in range(GRP):
                lane = jnp.zeros((L,), jnp.int32) + lax.bitwise_and(
                    svals[b], WIN - 1)
                assemble(rb, b, lane)
            pltpu.async_copy(rows_v.at[rb], out_h.at[dvx], sem2)
            return carry

        ngrp = (n + GRP - 1) // GRP
        lax.fori_loop(0, ngrp, group_body, 0)

        def drain_body(i, carry):
            pltpu.make_async_copy(out_h.at[dummy16], rows_v.at[0],
                                  sem2).wait()
            return carry

        lax.fori_loop(0, jnp.minimum(ngrp, 4), drain_body, 0)

    move_rows(mj_v, mo_v, val_t, nm)
    move_rows(us_v, uo_v, mem_t, nu)

    pltpu.async_copy(tail_t, tb_v.at[0], sem).wait()

    def tail_body(i, carry):
        e = (N_SAMPLE - 1) - i
        ch = lax.shift_right_logical(e, 4)
        ln = lax.bitwise_and(e, L - 1)
        sv = us_v[pl.ds(ch * L, L)]
        dvv = uo_v[pl.ds(ch * L, L)]
        s_spl = _bcast(sv, ln)
        d_spl = _bcast(dvv, ln)
        lane = s_spl - TAILSTART
        bsel = jnp.zeros((L,), jnp.int32)
        g0 = plsc.load_gather(tb_v, [bsel, iota, lane])
        g1 = plsc.load_gather(tb_v, [bsel, iota + L, lane])
        rows_v[0, 0, pl.ds(0, L)] = g0
        rows_v[0, 0, pl.ds(L, L)] = g1
        dvx = jnp.where(iota == 0, d_spl, DUMMY + iota)
        pltpu.async_copy(rows_v.at[0], out_h.at[dvx], sem2).wait()
        return carry

    lax.fori_loop(0, nt, tail_body, 0)


_sc_call_cache = []


def _get_sc_call():
    if not _sc_call_cache:
        _sc_call_cache.append(_build_sc_call())
    return _sc_call_cache[0]


def _build_sc_call():
    return pl.kernel(
        _body,
        out_type=jax.ShapeDtypeStruct((N_SAMPLE + 2 * L, 128), jnp.float32),
        mesh=plsc.VectorSubcoreMesh(core_axis_name="c", subcore_axis_name="s"),
        compiler_params=pltpu.CompilerParams(needs_layout_passes=False),
        scratch_types=[
            pltpu.VMEM((N_ADD,), jnp.int32),
            pltpu.VMEM((N_SAMPLE,), jnp.int32),
            pltpu.VMEM((TBL,), jnp.int32),
            pltpu.VMEM((N_ADD + L,), jnp.int32),
            pltpu.VMEM((LIST,), jnp.int32),
            pltpu.VMEM((LIST,), jnp.int32),
            pltpu.VMEM((LIST,), jnp.int32),
            pltpu.VMEM((LIST,), jnp.int32),
            pltpu.VMEM((GRP, DIM, WIN), jnp.float32),
            pltpu.VMEM((4, L, 128), jnp.float32),
            pltpu.SemaphoreType.DMA,
            pltpu.SemaphoreType.DMA,
        ],
    )


def kernel(mem, idx, val, sample_idx):
    mem_t = mem.T
    tail_t = mem_t[:, TAILSTART:]
    out = _get_sc_call()(mem_t, idx, val.T, sample_idx, tail_t)
    return out[:N_SAMPLE, :DIM]
